# Initial kernel scaffold; baseline (speedup 1.0000x reference)
import functools

import jax
import jax.numpy as jnp
from jax import lax
from jax.experimental import pallas as pl
from jax.experimental.pallas import tpu as pltpu
from jax.experimental.pallas import tpu_sc as plsc

NROW = 10000
NPAD = 10240
NC = 2
NS = 16
K = 128
E_PAD = 323584
RPT = NPAD // NS
D1 = 256
D2 = 128
RB = 512
NRB = NPAD // RB

_mesh = plsc.VectorSubcoreMesh(
    core_axis_name="c", subcore_axis_name="s", num_cores=NC, num_subcores=NS
)


@functools.partial(
    pl.kernel,
    out_type=jax.ShapeDtypeStruct((NC, NPAD, 16), jnp.float32),
    mesh=_mesh,
    scratch_types=[
        pltpu.VMEM((K,), jnp.int32),
        pltpu.VMEM((K, 16), jnp.float32),
        pltpu.VMEM_SHARED((NPAD, 16), jnp.float32),
    ],
)
def _deg_kernel(dst_hbm, zeros_hbm, ones_hbm, out_hbm, idx_v, ones_v, acc_sh):
    c = lax.axis_index("c")
    s = lax.axis_index("s")
    t0 = s * RPT
    pltpu.sync_copy(zeros_hbm, acc_sh.at[pl.ds(t0, RPT)])
    pltpu.sync_copy(ones_hbm, ones_v)
    plsc.subcore_barrier()
    epw = E_PAD // (NC * NS)
    base = (c * NS + s) * epw

    def body(g, carry):
        off = base + g * K
        pltpu.sync_copy(dst_hbm.at[pl.ds(off, K)], idx_v)
        pltpu.sync_copy(ones_v, acc_sh.at[idx_v], add=True)
        return carry

    lax.fori_loop(0, epw // K, body, 0)
    plsc.subcore_barrier()
    pltpu.sync_copy(acc_sh.at[pl.ds(t0, RPT)], out_hbm.at[c, pl.ds(t0, RPT)])


def _make_prop(d_sc):
    ept = E_PAD // NS
    nch = ept // K

    @functools.partial(
        pl.kernel,
        out_type=jax.ShapeDtypeStruct((NC, NPAD, d_sc), jnp.float32),
        mesh=_mesh,
        scratch_types=[
            pltpu.VMEM((K,), jnp.int32),
            pltpu.VMEM((K,), jnp.int32),
            pltpu.VMEM((K, d_sc), jnp.float32),
            pltpu.SemaphoreType.DMA,
            pltpu.VMEM_SHARED((NPAD, d_sc), jnp.float32),
        ],
    )
    def prop(xws_hbm, src2_hbm, dst_hbm, out_hbm, sidx, didx, rows_v, sem, acc_sh):
        c = lax.axis_index("c")
        s = lax.axis_index("s")
        t0 = s * RPT
        pltpu.sync_copy(
            xws_hbm.at[pl.ds(c * NPAD + t0, RPT)], acc_sh.at[pl.ds(t0, RPT)]
        )
        plsc.subcore_barrier()
        base = s * ept

        def body(g, carry):
            off = base + g * K
            pltpu.sync_copy(src2_hbm.at[c, pl.ds(off, K)], sidx)
            pltpu.sync_copy(dst_hbm.at[pl.ds(off, K)], didx)
            pltpu.async_copy(xws_hbm.at[sidx], rows_v, sem).wait()
            pltpu.sync_copy(rows_v, acc_sh.at[didx], add=True)
            return carry

        lax.fori_loop(0, nch, body, 0)
        plsc.subcore_barrier()
        pltpu.sync_copy(acc_sh.at[pl.ds(t0, RPT)], out_hbm.at[c, pl.ds(t0, RPT)])

    return prop


_prop_l1 = _make_prop(D1 // NC)
_prop_l2 = _make_prop(D2 // NC)


def _tc1_body(x_ref, w_ref, degp_ref, xws_ref, dinv_ref):
    deg = degp_ref[0, :, :1] + degp_ref[1, :, :1] + 1.0
    dinv = lax.rsqrt(deg)
    xw = jnp.dot(x_ref[...], w_ref[...], preferred_element_type=jnp.float32)
    xws_ref[0] = xw * dinv
    dinv_ref[...] = dinv


_tc1 = pl.pallas_call(
    _tc1_body,
    grid=(NRB, NC),
    in_specs=[
        pl.BlockSpec((RB, 128), lambda i, c: (i, 0)),
        pl.BlockSpec((128, D1 // NC), lambda i, c: (0, c)),
        pl.BlockSpec((NC, RB, 16), lambda i, c: (0, i, 0)),
    ],
    out_specs=[
        pl.BlockSpec((1, RB, D1 // NC), lambda i, c: (c, i, 0)),
        pl.BlockSpec((RB, 1), lambda i, c: (i, 0)),
    ],
    out_shape=[
        jax.ShapeDtypeStruct((NC, NPAD, D1 // NC), jnp.float32),
        jax.ShapeDtypeStruct((NPAD, 1), jnp.float32),
    ],
)


def _tc2_body(acc_ref, dinv_ref, b1_ref, w2_ref, xws_ref):
    acc = acc_ref[...]
    h = jnp.concatenate([acc[0], acc[1]], axis=1)
    h = jnp.maximum(h * dinv_ref[...] + b1_ref[...], 0.0)
    xw2 = jnp.dot(h, w2_ref[...], preferred_element_type=jnp.float32)
    xws_ref[0] = xw2 * dinv_ref[...]


_tc2 = pl.pallas_call(
    _tc2_body,
    grid=(NRB, NC),
    in_specs=[
        pl.BlockSpec((NC, RB, D1 // NC), lambda i, c: (0, i, 0)),
        pl.BlockSpec((RB, 1), lambda i, c: (i, 0)),
        pl.BlockSpec((1, D1), lambda i, c: (0, 0)),
        pl.BlockSpec((D1, D2 // NC), lambda i, c: (0, c)),
    ],
    out_specs=[pl.BlockSpec((1, RB, D2 // NC), lambda i, c: (c, i, 0))],
    out_shape=[jax.ShapeDtypeStruct((NC, NPAD, D2 // NC), jnp.float32)],
)


def _tc3_body(acc_ref, dinv_ref, b2_ref, out_ref):
    acc = acc_ref[...]
    z = jnp.concatenate([acc[0], acc[1]], axis=1)
    z = z * dinv_ref[...] + b2_ref[...]
    m = jnp.max(z, axis=1, keepdims=True)
    zs = z - m
    lse = jnp.log(jnp.sum(jnp.exp(zs), axis=1, keepdims=True))
    out_ref[...] = zs - lse


_tc3 = pl.pallas_call(
    _tc3_body,
    grid=(NRB,),
    in_specs=[
        pl.BlockSpec((NC, RB, D2 // NC), lambda i: (0, i, 0)),
        pl.BlockSpec((RB, 1), lambda i: (i, 0)),
        pl.BlockSpec((1, D2), lambda i: (0, 0)),
    ],
    out_specs=pl.BlockSpec((RB, D2), lambda i: (i, 0)),
    out_shape=jax.ShapeDtypeStruct((NPAD, D2), jnp.float32),
)


@jax.jit
def kernel(feature_data, edge_info, W1, b1, W2, b2):
    f32 = jnp.float32
    x = feature_data.astype(f32)
    src = edge_info[0].astype(jnp.int32)
    dst = edge_info[1].astype(jnp.int32)
    e = src.shape[0]
    pad = E_PAD - e
    src_p = jnp.concatenate([src, jnp.full((pad,), NROW, jnp.int32)])
    dst_p = jnp.concatenate([dst, jnp.full((pad,), NROW, jnp.int32)])
    src2 = jnp.stack([src_p, src_p + NPAD])
    x_pad = jnp.pad(x, ((0, NPAD - NROW), (0, 0)))

    zeros_tile = jnp.zeros((RPT, 16), f32)
    ones_chunk = jnp.ones((K, 16), f32)
    degp = _deg_kernel(dst_p, zeros_tile, ones_chunk)

    xws1, dinv = _tc1(x_pad, W1, degp)
    acc1 = _prop_l1(xws1.reshape(NC * NPAD, D1 // NC), src2, dst_p)
    (xws2,) = _tc2(acc1, dinv, b1.reshape(1, D1), W2)
    acc2 = _prop_l2(xws2.reshape(NC * NPAD, D2 // NC), src2, dst_p)
    out = _tc3(acc2, dinv, b2.reshape(1, D2))
    return out[:NROW]


# speedup vs baseline: 9.0404x; 9.0404x over previous
import functools

import jax
import jax.numpy as jnp
from jax import lax
from jax.experimental import pallas as pl
from jax.experimental.pallas import tpu as pltpu
from jax.experimental.pallas import tpu_sc as plsc

NROW = 10000
NPAD = 10240
NC = 2
NS = 16
K = 128
E_PAD = 323584
RPT = NPAD // NS
D1 = 256
D2 = 128
RB = 512
NRB = NPAD // RB

_mesh = plsc.VectorSubcoreMesh(
    core_axis_name="c", subcore_axis_name="s", num_cores=NC, num_subcores=NS
)


@functools.partial(
    pl.kernel,
    out_type=jax.ShapeDtypeStruct((NC, NPAD, 128), jnp.float32),
    mesh=_mesh,
    scratch_types=[
        pltpu.VMEM((K,), jnp.int32),
        pltpu.VMEM((K, 128), jnp.float32),
        pltpu.VMEM_SHARED((NPAD, 128), jnp.float32),
    ],
)
def _deg_kernel(dst_hbm, zeros_hbm, ones_hbm, out_hbm, idx_v, ones_v, acc_sh):
    c = lax.axis_index("c")
    s = lax.axis_index("s")
    t0 = s * RPT
    pltpu.sync_copy(zeros_hbm, acc_sh.at[pl.ds(t0, RPT)])
    pltpu.sync_copy(ones_hbm, ones_v)
    plsc.subcore_barrier()
    epw = E_PAD // (NC * NS)
    base = (c * NS + s) * epw

    def body(g, carry):
        off = base + g * K
        pltpu.sync_copy(dst_hbm.at[pl.ds(off, K)], idx_v)
        pltpu.sync_copy(ones_v, acc_sh.at[idx_v], add=True)
        return carry

    lax.fori_loop(0, epw // K, body, 0)
    plsc.subcore_barrier()
    pltpu.sync_copy(acc_sh.at[pl.ds(t0, RPT)], out_hbm.at[c, pl.ds(t0, RPT)])


@functools.partial(
    pl.kernel,
    out_type=jax.ShapeDtypeStruct((NC, NPAD, D1 // NC), jnp.float32),
    mesh=_mesh,
    scratch_types=[
        pltpu.VMEM((K,), jnp.int32),
        pltpu.VMEM((K,), jnp.int32),
        pltpu.VMEM((K, D1 // NC), jnp.float32),
        pltpu.SemaphoreType.DMA,
        pltpu.VMEM_SHARED((NPAD, D1 // NC), jnp.float32),
    ],
)
def _prop_l1(xws_hbm, src2_hbm, dst_hbm, out_hbm, sidx, didx, rows_v, sem,
             acc_sh):
    c = lax.axis_index("c")
    s = lax.axis_index("s")
    t0 = s * RPT
    pltpu.sync_copy(
        xws_hbm.at[pl.ds(c * NPAD + t0, RPT)], acc_sh.at[pl.ds(t0, RPT)]
    )
    plsc.subcore_barrier()
    ept = E_PAD // NS
    base = s * ept

    def body(g, carry):
        off = base + g * K
        pltpu.sync_copy(src2_hbm.at[c, pl.ds(off, K)], sidx)
        pltpu.sync_copy(dst_hbm.at[pl.ds(off, K)], didx)
        pltpu.async_copy(xws_hbm.at[sidx], rows_v, sem).wait()
        pltpu.sync_copy(rows_v, acc_sh.at[didx], add=True)
        return carry

    lax.fori_loop(0, ept // K, body, 0)
    plsc.subcore_barrier()
    pltpu.sync_copy(acc_sh.at[pl.ds(t0, RPT)], out_hbm.at[c, pl.ds(t0, RPT)])


@functools.partial(
    pl.kernel,
    out_type=jax.ShapeDtypeStruct((NC, NPAD, D2), jnp.float32),
    mesh=_mesh,
    scratch_types=[
        pltpu.VMEM((K,), jnp.int32),
        pltpu.VMEM((K,), jnp.int32),
        pltpu.VMEM((K, D2), jnp.float32),
        pltpu.SemaphoreType.DMA,
        pltpu.VMEM_SHARED((NPAD, D2), jnp.float32),
    ],
)
def _prop_l2(xws_hbm, zeros_hbm, src_hbm, dst_hbm, out_hbm, sidx, didx, rows_v,
             sem, acc_sh):
    c = lax.axis_index("c")
    s = lax.axis_index("s")
    t0 = s * RPT
    pltpu.sync_copy(zeros_hbm, acc_sh.at[pl.ds(t0, RPT)])
    plsc.subcore_barrier()
    epw = E_PAD // (NC * NS)
    base = (c * NS + s) * epw

    def body(g, carry):
        off = base + g * K
        pltpu.sync_copy(src_hbm.at[pl.ds(off, K)], sidx)
        pltpu.sync_copy(dst_hbm.at[pl.ds(off, K)], didx)
        pltpu.async_copy(xws_hbm.at[sidx], rows_v, sem).wait()
        pltpu.sync_copy(rows_v, acc_sh.at[didx], add=True)
        return carry

    lax.fori_loop(0, epw // K, body, 0)
    plsc.subcore_barrier()
    pltpu.sync_copy(acc_sh.at[pl.ds(t0, RPT)], out_hbm.at[c, pl.ds(t0, RPT)])


def _tc1_body(x_ref, w_ref, degp_ref, xws_ref, dinv_ref):
    deg = degp_ref[0, :, :1] + degp_ref[1, :, :1] + 1.0
    dinv = lax.rsqrt(deg)
    xw = jnp.dot(x_ref[...], w_ref[...], preferred_element_type=jnp.float32)
    xws_ref[0] = xw * dinv
    dinv_ref[...] = dinv


_tc1 = pl.pallas_call(
    _tc1_body,
    grid=(NRB, NC),
    in_specs=[
        pl.BlockSpec((RB, 128), lambda i, c: (i, 0)),
        pl.BlockSpec((128, D1 // NC), lambda i, c: (0, c)),
        pl.BlockSpec((NC, RB, 128), lambda i, c: (0, i, 0)),
    ],
    out_specs=[
        pl.BlockSpec((1, RB, D1 // NC), lambda i, c: (c, i, 0)),
        pl.BlockSpec((RB, 1), lambda i, c: (i, 0)),
    ],
    out_shape=[
        jax.ShapeDtypeStruct((NC, NPAD, D1 // NC), jnp.float32),
        jax.ShapeDtypeStruct((NPAD, 1), jnp.float32),
    ],
)


def _tc2_body(acc_ref, dinv_ref, b1_ref, w2_ref, xws_ref):
    acc = acc_ref[...]
    h = jnp.concatenate([acc[0], acc[1]], axis=1)
    h = jnp.maximum(h * dinv_ref[...] + b1_ref[...], 0.0)
    xw2 = jnp.dot(h, w2_ref[...], preferred_element_type=jnp.float32)
    xws_ref[...] = xw2 * dinv_ref[...]


_tc2 = pl.pallas_call(
    _tc2_body,
    grid=(NRB,),
    in_specs=[
        pl.BlockSpec((NC, RB, D1 // NC), lambda i: (0, i, 0)),
        pl.BlockSpec((RB, 1), lambda i: (i, 0)),
        pl.BlockSpec((1, D1), lambda i: (0, 0)),
        pl.BlockSpec((D1, D2), lambda i: (0, 0)),
    ],
    out_specs=[pl.BlockSpec((RB, D2), lambda i: (i, 0))],
    out_shape=[jax.ShapeDtypeStruct((NPAD, D2), jnp.float32)],
)


def _tc3_body(acc_ref, xws2_ref, dinv_ref, b2_ref, out_ref):
    acc = acc_ref[...]
    z = acc[0] + acc[1] + xws2_ref[...]
    z = z * dinv_ref[...] + b2_ref[...]
    m = jnp.max(z, axis=1, keepdims=True)
    zs = z - m
    lse = jnp.log(jnp.sum(jnp.exp(zs), axis=1, keepdims=True))
    out_ref[...] = zs - lse


_tc3 = pl.pallas_call(
    _tc3_body,
    grid=(NRB,),
    in_specs=[
        pl.BlockSpec((NC, RB, D2), lambda i: (0, i, 0)),
        pl.BlockSpec((RB, D2), lambda i: (i, 0)),
        pl.BlockSpec((RB, 1), lambda i: (i, 0)),
        pl.BlockSpec((1, D2), lambda i: (0, 0)),
    ],
    out_specs=pl.BlockSpec((RB, D2), lambda i: (i, 0)),
    out_shape=jax.ShapeDtypeStruct((NPAD, D2), jnp.float32),
)


@jax.jit
def kernel(feature_data, edge_info, W1, b1, W2, b2):
    f32 = jnp.float32
    x = feature_data.astype(f32)
    src = edge_info[0].astype(jnp.int32)
    dst = edge_info[1].astype(jnp.int32)
    e = src.shape[0]
    pad = E_PAD - e
    src_p = jnp.concatenate([src, jnp.full((pad,), NROW, jnp.int32)])
    dst_p = jnp.concatenate([dst, jnp.full((pad,), NROW, jnp.int32)])
    src2 = jnp.stack([src_p, src_p + NPAD])
    x_pad = jnp.pad(x, ((0, NPAD - NROW), (0, 0)))

    zeros_row = jnp.zeros((RPT, D2), f32)
    ones_chunk = jnp.ones((K, 128), f32)
    degp = _deg_kernel(dst_p, zeros_row, ones_chunk)

    xws1, dinv = _tc1(x_pad, W1, degp)
    acc1 = _prop_l1(xws1.reshape(NC * NPAD, D1 // NC), src2, dst_p)
    (xws2,) = _tc2(acc1, dinv, b1.reshape(1, D1), W2)
    acc2 = _prop_l2(xws2, zeros_row, src_p, dst_p)
    out = _tc3(acc2, xws2, dinv, b2.reshape(1, D2))
    return out[:NROW]


# speedup vs baseline: 9.3277x; 1.0318x over previous
import functools

import jax
import jax.numpy as jnp
from jax import lax
from jax.experimental import pallas as pl
from jax.experimental.pallas import tpu as pltpu
from jax.experimental.pallas import tpu_sc as plsc

NROW = 10000
NPAD = 10240
NC = 2
NS = 16
K = 128
E_PAD = 323584
RPT = NPAD // NS
D1 = 256
D2 = 128
RB = 512
NRB = NPAD // RB

_mesh = plsc.VectorSubcoreMesh(
    core_axis_name="c", subcore_axis_name="s", num_cores=NC, num_subcores=NS
)


@functools.partial(
    pl.kernel,
    out_type=jax.ShapeDtypeStruct((NC, NPAD, 128), jnp.float32),
    mesh=_mesh,
    scratch_types=[
        pltpu.VMEM((K,), jnp.int32),
        pltpu.VMEM((K, 128), jnp.float32),
        pltpu.VMEM_SHARED((NPAD, 128), jnp.float32),
    ],
)
def _deg_kernel(dst_hbm, zeros_hbm, ones_hbm, out_hbm, idx_v, ones_v, acc_sh):
    c = lax.axis_index("c")
    s = lax.axis_index("s")
    t0 = s * RPT
    pltpu.sync_copy(zeros_hbm, acc_sh.at[pl.ds(t0, RPT)])
    pltpu.sync_copy(ones_hbm, ones_v)
    plsc.subcore_barrier()
    epw = E_PAD // (NC * NS)
    base = (c * NS + s) * epw

    def body(g, carry):
        off = base + g * K
        pltpu.sync_copy(dst_hbm.at[pl.ds(off, K)], idx_v)
        pltpu.sync_copy(ones_v, acc_sh.at[idx_v], add=True)
        return carry

    lax.fori_loop(0, epw // K, body, 0)
    plsc.subcore_barrier()
    pltpu.sync_copy(acc_sh.at[pl.ds(t0, RPT)], out_hbm.at[c, pl.ds(t0, RPT)])


@functools.partial(
    pl.kernel,
    out_type=jax.ShapeDtypeStruct((NC, NPAD, D1 // NC), jnp.float32),
    mesh=_mesh,
    scratch_types=[
        pltpu.VMEM((2, K), jnp.int32),
        pltpu.VMEM((K, D1 // NC), jnp.float32),
        pltpu.SemaphoreType.DMA,
        pltpu.VMEM_SHARED((NPAD, D1 // NC), jnp.float32),
    ],
)
def _prop_l1(xws_hbm, edges_hbm, out_hbm, idx2, rows_v, sem, acc_sh):
    c = lax.axis_index("c")
    s = lax.axis_index("s")
    t0 = s * RPT
    pltpu.sync_copy(
        xws_hbm.at[pl.ds(c * NPAD + t0, RPT)], acc_sh.at[pl.ds(t0, RPT)]
    )
    plsc.subcore_barrier()
    nch = E_PAD // K // NS
    base = s * nch

    def body(g, carry):
        pltpu.sync_copy(edges_hbm.at[c, base + g], idx2)
        pltpu.async_copy(xws_hbm.at[idx2.at[0]], rows_v, sem).wait()
        pltpu.sync_copy(rows_v, acc_sh.at[idx2.at[1]], add=True)
        return carry

    lax.fori_loop(0, nch, body, 0)
    plsc.subcore_barrier()
    pltpu.sync_copy(acc_sh.at[pl.ds(t0, RPT)], out_hbm.at[c, pl.ds(t0, RPT)])


@functools.partial(
    pl.kernel,
    out_type=jax.ShapeDtypeStruct((NC, NPAD, D2), jnp.float32),
    mesh=_mesh,
    scratch_types=[
        pltpu.VMEM((2, K), jnp.int32),
        pltpu.VMEM((K, D2), jnp.float32),
        pltpu.SemaphoreType.DMA,
        pltpu.VMEM_SHARED((NPAD, D2), jnp.float32),
    ],
)
def _prop_l2(xws_hbm, zeros_hbm, edges_hbm, out_hbm, idx2, rows_v, sem, acc_sh):
    c = lax.axis_index("c")
    s = lax.axis_index("s")
    t0 = s * RPT
    pltpu.sync_copy(zeros_hbm, acc_sh.at[pl.ds(t0, RPT)])
    plsc.subcore_barrier()
    nch = E_PAD // K // (NC * NS)
    base = (c * NS + s) * nch

    def body(g, carry):
        pltpu.sync_copy(edges_hbm.at[base + g], idx2)
        pltpu.async_copy(xws_hbm.at[idx2.at[0]], rows_v, sem).wait()
        pltpu.sync_copy(rows_v, acc_sh.at[idx2.at[1]], add=True)
        return carry

    lax.fori_loop(0, nch, body, 0)
    plsc.subcore_barrier()
    pltpu.sync_copy(acc_sh.at[pl.ds(t0, RPT)], out_hbm.at[c, pl.ds(t0, RPT)])


def _tc1_body(x_ref, w_ref, degp_ref, xws_ref, dinv_ref):
    deg = degp_ref[0, :, :1] + degp_ref[1, :, :1] + 1.0
    dinv = lax.rsqrt(deg)
    xw = jnp.dot(x_ref[...], w_ref[...], preferred_element_type=jnp.float32)
    xws_ref[0] = xw * dinv
    dinv_ref[...] = dinv


_tc1 = pl.pallas_call(
    _tc1_body,
    grid=(NRB, NC),
    in_specs=[
        pl.BlockSpec((RB, 128), lambda i, c: (i, 0)),
        pl.BlockSpec((128, D1 // NC), lambda i, c: (0, c)),
        pl.BlockSpec((NC, RB, 128), lambda i, c: (0, i, 0)),
    ],
    out_specs=[
        pl.BlockSpec((1, RB, D1 // NC), lambda i, c: (c, i, 0)),
        pl.BlockSpec((RB, 1), lambda i, c: (i, 0)),
    ],
    out_shape=[
        jax.ShapeDtypeStruct((NC, NPAD, D1 // NC), jnp.float32),
        jax.ShapeDtypeStruct((NPAD, 1), jnp.float32),
    ],
)


def _tc2_body(acc_ref, dinv_ref, b1_ref, w2_ref, xws_ref):
    acc = acc_ref[...]
    h = jnp.concatenate([acc[0], acc[1]], axis=1)
    h = jnp.maximum(h * dinv_ref[...] + b1_ref[...], 0.0)
    xw2 = jnp.dot(h, w2_ref[...], preferred_element_type=jnp.float32)
    xws_ref[...] = xw2 * dinv_ref[...]


_tc2 = pl.pallas_call(
    _tc2_body,
    grid=(NRB,),
    in_specs=[
        pl.BlockSpec((NC, RB, D1 // NC), lambda i: (0, i, 0)),
        pl.BlockSpec((RB, 1), lambda i: (i, 0)),
        pl.BlockSpec((1, D1), lambda i: (0, 0)),
        pl.BlockSpec((D1, D2), lambda i: (0, 0)),
    ],
    out_specs=[pl.BlockSpec((RB, D2), lambda i: (i, 0))],
    out_shape=[jax.ShapeDtypeStruct((NPAD, D2), jnp.float32)],
)


def _tc3_body(acc_ref, xws2_ref, dinv_ref, b2_ref, out_ref):
    acc = acc_ref[...]
    z = acc[0] + acc[1] + xws2_ref[...]
    z = z * dinv_ref[...] + b2_ref[...]
    m = jnp.max(z, axis=1, keepdims=True)
    zs = z - m
    lse = jnp.log(jnp.sum(jnp.exp(zs), axis=1, keepdims=True))
    out_ref[...] = zs - lse


_tc3 = pl.pallas_call(
    _tc3_body,
    grid=(NRB,),
    in_specs=[
        pl.BlockSpec((NC, RB, D2), lambda i: (0, i, 0)),
        pl.BlockSpec((RB, D2), lambda i: (i, 0)),
        pl.BlockSpec((RB, 1), lambda i: (i, 0)),
        pl.BlockSpec((1, D2), lambda i: (0, 0)),
    ],
    out_specs=pl.BlockSpec((RB, D2), lambda i: (i, 0)),
    out_shape=jax.ShapeDtypeStruct((NPAD, D2), jnp.float32),
)


@jax.jit
def kernel(feature_data, edge_info, W1, b1, W2, b2):
    f32 = jnp.float32
    x = feature_data.astype(f32)
    src = edge_info[0].astype(jnp.int32)
    dst = edge_info[1].astype(jnp.int32)
    e = src.shape[0]
    pad = E_PAD - e
    src_p = jnp.concatenate([src, jnp.full((pad,), NROW, jnp.int32)])
    dst_p = jnp.concatenate([dst, jnp.full((pad,), NROW, jnp.int32)])
    srcg = src_p.reshape(E_PAD // K, K)
    dstg = dst_p.reshape(E_PAD // K, K)
    edges1 = jnp.stack(
        [jnp.stack([srcg, dstg], axis=1),
         jnp.stack([srcg + NPAD, dstg], axis=1)])
    edges2 = jnp.stack([srcg, dstg], axis=1)
    x_pad = jnp.pad(x, ((0, NPAD - NROW), (0, 0)))

    zeros_row = jnp.zeros((RPT, D2), f32)
    ones_chunk = jnp.ones((K, 128), f32)
    degp = _deg_kernel(dst_p, zeros_row, ones_chunk)

    xws1, dinv = _tc1(x_pad, W1, degp)
    acc1 = _prop_l1(xws1.reshape(NC * NPAD, D1 // NC), edges1)
    (xws2,) = _tc2(acc1, dinv, b1.reshape(1, D1), W2)
    acc2 = _prop_l2(xws2, zeros_row, edges2)
    out = _tc3(acc2, xws2, dinv, b2.reshape(1, D2))
    return out[:NROW]
